# baseline (speedup 1.0000x reference)
import functools

import jax
import jax.numpy as jnp
from jax import lax
from jax.experimental import pallas as pl
from jax.experimental.pallas import tpu as pltpu
from jax.experimental.pallas import tpu_sc as plsc


def _make_gather(N, S, D, NB):
    info = plsc.get_sparse_core_info()
    NC, NS = info.num_cores, info.num_subcores
    NW = NC * NS
    rows_per_w = N // NW
    n_groups = rows_per_w // NB
    assert N % NW == 0 and rows_per_w % NB == 0

    mesh = plsc.VectorSubcoreMesh(core_axis_name="c", subcore_axis_name="s")

    @functools.partial(
        pl.kernel,
        out_type=jax.ShapeDtypeStruct((N, S, D), jnp.float32),
        mesh=mesh,
        scratch_types=[
            pltpu.VMEM((rows_per_w, S), jnp.int32),
            pltpu.VMEM((NB, S, D), jnp.float32),
            pltpu.SemaphoreType.DMA((NB,)),
            pltpu.SemaphoreType.DMA((NB,)),
        ],
        compiler_params=pltpu.CompilerParams(use_tc_tiling_on_sc=False),
    )
    def gather(idx_hbm, table_hbm, out_hbm, idx_v, rows_v, gsem, wsem):
        wid = lax.axis_index("s") * NC + lax.axis_index("c")
        row0 = wid * rows_per_w
        pltpu.sync_copy(idx_hbm.at[pl.ds(row0, rows_per_w)], idx_v)

        def gather_copy(i, b):
            return pltpu.make_async_copy(
                table_hbm.at[idx_v.at[i]], rows_v.at[b], gsem.at[b]
            )

        def wb_copy(i, b):
            return pltpu.make_async_copy(
                rows_v.at[b], out_hbm.at[row0 + i], wsem.at[b]
            )

        for b in range(NB):
            gather_copy(b, b).start()

        @pl.loop(1, n_groups)
        def _(g):
            i0 = g * NB
            for b in range(NB):
                prev = i0 - NB + b
                gather_copy(prev, b).wait()
                wb_copy(prev, b).start()
            for b in range(NB):
                wb_copy(i0 - NB + b, b).wait()
                gather_copy(i0 + b, b).start()

        last0 = (n_groups - 1) * NB
        for b in range(NB):
            gather_copy(last0 + b, b).wait()
            wb_copy(last0 + b, b).start()
        for b in range(NB):
            wb_copy(last0 + b, b).wait()

    return gather


def kernel(indices, wte):
    n, s = indices.shape
    _, D = wte.shape
    gather = _make_gather(n, s, D, NB=4)
    return gather(indices, wte)
